# baseline (device time: 172179 ns/iter reference)
import jax
import jax.numpy as jnp
from jax import lax
from jax.experimental import pallas as pl
from jax.experimental.pallas import tpu as pltpu

N_DEV = 4
N_TOK = 2048
D_IN = 512
D_OUT = 1024
N_EXP = 16
EXP_PER_DEV = N_EXP // N_DEV
CHUNK = N_TOK // N_DEV


def kernel(x, router_W, route_idx, expert_W):
    def body(x_ref, rw_ref, idx_ref, ew_ref, out_ref,
             acc_ref, rs_recv_ref, send_sems, recv_sems):
        my = lax.axis_index("i")
        left = (my + N_DEV - 1) % N_DEV
        right = (my + 1) % N_DEV

        barrier_sem = pltpu.get_barrier_semaphore()
        for nbr in (left, right):
            pl.semaphore_signal(
                barrier_sem, inc=1,
                device_id=(nbr,), device_id_type=pl.DeviceIdType.MESH,
            )
        pl.semaphore_wait(barrier_sem, 2)

        x_val = x_ref[:, :]
        scores = jnp.dot(x_val, rw_ref[:, :], preferred_element_type=jnp.float32)
        m = jnp.max(scores, axis=1, keepdims=True)
        e = jnp.exp(scores - m)
        probs = e / jnp.sum(e, axis=1, keepdims=True)

        e_ids = lax.broadcasted_iota(jnp.int32, (N_TOK, N_EXP), 1)
        mask = jnp.logical_or(
            e_ids == idx_ref[:, 0:1], e_ids == idx_ref[:, 1:2]
        ).astype(jnp.float32)
        masked = probs * mask
        gates = masked / jnp.sum(masked, axis=1, keepdims=True)

        sel = (
            lax.broadcasted_iota(jnp.int32, (N_EXP, EXP_PER_DEV), 0)
            == EXP_PER_DEV * my
            + lax.broadcasted_iota(jnp.int32, (N_EXP, EXP_PER_DEV), 1)
        ).astype(jnp.float32)
        g_local = jnp.dot(gates, sel, preferred_element_type=jnp.float32)

        acc_ref[:, :] = g_local[:, 0:1] * jnp.dot(
            x_val, ew_ref[0], preferred_element_type=jnp.float32
        )
        for j in range(1, EXP_PER_DEV):
            acc_ref[:, :] += g_local[:, j : j + 1] * jnp.dot(
                x_val, ew_ref[j], preferred_element_type=jnp.float32
            )

        for s in range(N_DEV - 1):
            send_c = (my + N_DEV - s) % N_DEV
            recv_c = (my + N_DEV - 1 - s) % N_DEV
            rdma = pltpu.make_async_remote_copy(
                src_ref=acc_ref.at[pl.ds(send_c * CHUNK, CHUNK), :],
                dst_ref=rs_recv_ref.at[s],
                send_sem=send_sems.at[s],
                recv_sem=recv_sems.at[s],
                device_id=(right,),
                device_id_type=pl.DeviceIdType.MESH,
            )
            rdma.start()
            rdma.wait()
            acc_ref[pl.ds(recv_c * CHUNK, CHUNK), :] = (
                acc_ref[pl.ds(recv_c * CHUNK, CHUNK), :] + rs_recv_ref[s]
            )

        mine_c = (my + 1) % N_DEV
        out_ref[pl.ds(mine_c * CHUNK, CHUNK), :] = acc_ref[
            pl.ds(mine_c * CHUNK, CHUNK), :
        ]

        for t in range(N_DEV - 1):
            send_c = (my + 1 + N_DEV - t) % N_DEV
            rdma = pltpu.make_async_remote_copy(
                src_ref=out_ref.at[pl.ds(send_c * CHUNK, CHUNK), :],
                dst_ref=out_ref.at[pl.ds(send_c * CHUNK, CHUNK), :],
                send_sem=send_sems.at[N_DEV - 1 + t],
                recv_sem=recv_sems.at[N_DEV - 1 + t],
                device_id=(right,),
                device_id_type=pl.DeviceIdType.MESH,
            )
            rdma.start()
            rdma.wait()

    return pl.pallas_call(
        body,
        out_shape=jax.ShapeDtypeStruct((N_TOK, D_OUT), jnp.float32),
        in_specs=[
            pl.BlockSpec(memory_space=pltpu.VMEM),
            pl.BlockSpec(memory_space=pltpu.VMEM),
            pl.BlockSpec(memory_space=pltpu.VMEM),
            pl.BlockSpec(memory_space=pltpu.VMEM),
        ],
        out_specs=pl.BlockSpec(memory_space=pltpu.VMEM),
        scratch_shapes=[
            pltpu.VMEM((N_TOK, D_OUT), jnp.float32),
            pltpu.VMEM((N_DEV - 1, CHUNK, D_OUT), jnp.float32),
            pltpu.SemaphoreType.DMA((2 * (N_DEV - 1),)),
            pltpu.SemaphoreType.DMA((2 * (N_DEV - 1),)),
        ],
        compiler_params=pltpu.CompilerParams(collective_id=0),
    )(x, router_W, route_idx, expert_W)


# device time: 101552 ns/iter; 1.6955x vs baseline; 1.6955x over previous
import jax
import jax.numpy as jnp
from jax import lax
from jax.experimental import pallas as pl
from jax.experimental.pallas import tpu as pltpu

N_DEV = 4
N_TOK = 2048
D_IN = 512
D_OUT = 1024
N_EXP = 16
EXP_PER_DEV = N_EXP // N_DEV
Q = N_TOK // N_DEV
HALF = D_OUT // 2


def kernel(x, router_W, route_idx, expert_W):
    def body(x_ref, rw_ref, idx_ref, ew_ref, out_ref,
             acc_ref, r1_ref, r2_ref, send_sems, recv_sems):
        my = lax.axis_index("i")
        pA = my ^ 1
        pB = 3 - my

        barrier_sem = pltpu.get_barrier_semaphore()
        for nbr in (pA, pB):
            pl.semaphore_signal(
                barrier_sem, inc=1,
                device_id=(nbr,), device_id_type=pl.DeviceIdType.MESH,
            )
        pl.semaphore_wait(barrier_sem, 2)

        x_val = x_ref[:, :]
        scores = jnp.dot(x_val, rw_ref[:, :], preferred_element_type=jnp.float32)
        m = jnp.max(scores, axis=1, keepdims=True)
        e = jnp.exp(scores - m)
        probs = e / jnp.sum(e, axis=1, keepdims=True)
        e_ids = lax.broadcasted_iota(jnp.int32, (N_TOK, N_EXP), 1)
        mask = jnp.logical_or(
            e_ids == idx_ref[:, 0:1], e_ids == idx_ref[:, 1:2]
        ).astype(jnp.float32)
        masked = probs * mask
        gates = masked / jnp.sum(masked, axis=1, keepdims=True)
        sel = (
            lax.broadcasted_iota(jnp.int32, (N_EXP, EXP_PER_DEV), 0)
            == EXP_PER_DEV * my
            + lax.broadcasted_iota(jnp.int32, (N_EXP, EXP_PER_DEV), 1)
        ).astype(jnp.float32)
        g_local = jnp.dot(gates, sel, preferred_element_type=jnp.float32)

        def rows(q):
            return pl.ds(q * Q, Q)

        def cols(h):
            return pl.ds(h * HALF, HALF)

        def compute_half(h):
            v = g_local[:, 0:1] * jnp.dot(
                x_val, ew_ref[0, :, h * HALF:(h + 1) * HALF],
                preferred_element_type=jnp.float32,
            )
            for j in range(1, EXP_PER_DEV):
                v += g_local[:, j:j + 1] * jnp.dot(
                    x_val, ew_ref[j, :, h * HALF:(h + 1) * HALF],
                    preferred_element_type=jnp.float32,
                )
            acc_ref[:, cols(h)] = v

        def half_plan(h):
            if h == 0:
                p1, p2 = pA, pB
                q4 = 3 - p1
            else:
                p1, p2 = pB, pA
                q4 = p1 ^ 1
            return p1, p2, (p1, q4)

        def start_rs1(h):
            p1, p2, send_qs = half_plan(h)
            rds = []
            for k, q in enumerate(send_qs):
                r = pltpu.make_async_remote_copy(
                    src_ref=acc_ref.at[rows(q), cols(h)],
                    dst_ref=r1_ref.at[h, k],
                    send_sem=send_sems.at[6 * h + k],
                    recv_sem=recv_sems.at[6 * h + k],
                    device_id=(p1,),
                    device_id_type=pl.DeviceIdType.MESH,
                )
                r.start()
                rds.append(r)
            return rds

        def finish_rs1_start_rs2(h, rds):
            p1, p2, _ = half_plan(h)
            for r in rds:
                r.wait()
            acc_ref[rows(my), cols(h)] += r1_ref[h, 0]
            acc_ref[rows(p2), cols(h)] += r1_ref[h, 1]
            r = pltpu.make_async_remote_copy(
                src_ref=acc_ref.at[rows(p2), cols(h)],
                dst_ref=r2_ref.at[h],
                send_sem=send_sems.at[6 * h + 2],
                recv_sem=recv_sems.at[6 * h + 2],
                device_id=(p2,),
                device_id_type=pl.DeviceIdType.MESH,
            )
            r.start()
            return [r]

        def finish_rs2_start_ag1(h, rds):
            p1, p2, _ = half_plan(h)
            for r in rds:
                r.wait()
            out_ref[rows(my), cols(h)] = (
                acc_ref[rows(my), cols(h)] + r2_ref[h]
            )
            r = pltpu.make_async_remote_copy(
                src_ref=out_ref.at[rows(my), cols(h)],
                dst_ref=out_ref.at[rows(my), cols(h)],
                send_sem=send_sems.at[6 * h + 3],
                recv_sem=recv_sems.at[6 * h + 3],
                device_id=(p2,),
                device_id_type=pl.DeviceIdType.MESH,
            )
            r.start()
            return [r]

        def finish_ag1_start_ag2(h, rds):
            p1, p2, _ = half_plan(h)
            for r in rds:
                r.wait()
            rds2 = []
            for k, q in enumerate((my, p2)):
                r = pltpu.make_async_remote_copy(
                    src_ref=out_ref.at[rows(q), cols(h)],
                    dst_ref=out_ref.at[rows(q), cols(h)],
                    send_sem=send_sems.at[6 * h + 4 + k],
                    recv_sem=recv_sems.at[6 * h + 4 + k],
                    device_id=(p1,),
                    device_id_type=pl.DeviceIdType.MESH,
                )
                r.start()
                rds2.append(r)
            return rds2

        compute_half(0)
        h0 = start_rs1(0)
        compute_half(1)
        h1 = start_rs1(1)
        h0 = finish_rs1_start_rs2(0, h0)
        h1 = finish_rs1_start_rs2(1, h1)
        h0 = finish_rs2_start_ag1(0, h0)
        h1 = finish_rs2_start_ag1(1, h1)
        h0 = finish_ag1_start_ag2(0, h0)
        h1 = finish_ag1_start_ag2(1, h1)
        for r in h0 + h1:
            r.wait()

    return pl.pallas_call(
        body,
        out_shape=jax.ShapeDtypeStruct((N_TOK, D_OUT), jnp.float32),
        in_specs=[
            pl.BlockSpec(memory_space=pltpu.VMEM),
            pl.BlockSpec(memory_space=pltpu.VMEM),
            pl.BlockSpec(memory_space=pltpu.VMEM),
            pl.BlockSpec(memory_space=pltpu.VMEM),
        ],
        out_specs=pl.BlockSpec(memory_space=pltpu.VMEM),
        scratch_shapes=[
            pltpu.VMEM((N_TOK, D_OUT), jnp.float32),
            pltpu.VMEM((2, 2, Q, HALF), jnp.float32),
            pltpu.VMEM((2, Q, HALF), jnp.float32),
            pltpu.SemaphoreType.DMA((12,)),
            pltpu.SemaphoreType.DMA((12,)),
        ],
        compiler_params=pltpu.CompilerParams(collective_id=0),
    )(x, router_W, route_idx, expert_W)


# device time: 96603 ns/iter; 1.7823x vs baseline; 1.0512x over previous
import jax
import jax.numpy as jnp
from jax import lax
from jax.experimental import pallas as pl
from jax.experimental.pallas import tpu as pltpu

N_DEV = 4
N_TOK = 2048
D_IN = 512
D_OUT = 1024
N_EXP = 16
EXP_PER_DEV = N_EXP // N_DEV
Q = N_TOK // N_DEV
HALF = D_OUT // 2


def kernel(x, router_W, route_idx, expert_W):
    def body(x_ref, rw_ref, idx_ref, ew_ref, out_ref,
             acc_ref, r1_ref, r2_ref, g_ref, send_sems, recv_sems):
        my = lax.axis_index("i")
        pA = my ^ 1
        pB = 3 - my

        barrier_sem = pltpu.get_barrier_semaphore()
        for nbr in (pA, pB):
            pl.semaphore_signal(
                barrier_sem, inc=1,
                device_id=(nbr,), device_id_type=pl.DeviceIdType.MESH,
            )
        pl.semaphore_wait(barrier_sem, 2)

        x_val = x_ref[:, :]
        scores = jnp.dot(x_val, rw_ref[:, :], preferred_element_type=jnp.float32)
        m = jnp.max(scores, axis=1, keepdims=True)
        e = jnp.exp(scores - m)
        probs = e / jnp.sum(e, axis=1, keepdims=True)
        e_ids = lax.broadcasted_iota(jnp.int32, (N_TOK, N_EXP), 1)
        mask = jnp.logical_or(
            e_ids == idx_ref[:, 0:1], e_ids == idx_ref[:, 1:2]
        ).astype(jnp.float32)
        masked = probs * mask
        gates = masked / jnp.sum(masked, axis=1, keepdims=True)
        sel = (
            lax.broadcasted_iota(jnp.int32, (N_EXP, EXP_PER_DEV), 0)
            == EXP_PER_DEV * my
            + lax.broadcasted_iota(jnp.int32, (N_EXP, EXP_PER_DEV), 1)
        ).astype(jnp.float32)
        g_ref[:, :] = jnp.dot(gates, sel, preferred_element_type=jnp.float32)

        def rows(q):
            return pl.ds(q * Q, Q)

        def cols(h):
            return pl.ds(h * HALF, HALF)

        def compute_quarters(h, qs):
            for q in qs:
                xq = x_ref[rows(q), :]
                gq = g_ref[rows(q), :]
                v = gq[:, 0:1] * jnp.dot(
                    xq, ew_ref[0, :, h * HALF:(h + 1) * HALF],
                    preferred_element_type=jnp.float32,
                )
                for j in range(1, EXP_PER_DEV):
                    v += gq[:, j:j + 1] * jnp.dot(
                        xq, ew_ref[j, :, h * HALF:(h + 1) * HALF],
                        preferred_element_type=jnp.float32,
                    )
                acc_ref[rows(q), cols(h)] = v

        def half_plan(h):
            if h == 0:
                p1, p2 = pA, pB
                q4 = 3 - p1
            else:
                p1, p2 = pB, pA
                q4 = p1 ^ 1
            return p1, p2, (p1, q4)

        def start_rs1(h):
            p1, p2, send_qs = half_plan(h)
            rds = []
            for k, q in enumerate(send_qs):
                r = pltpu.make_async_remote_copy(
                    src_ref=acc_ref.at[rows(q), cols(h)],
                    dst_ref=r1_ref.at[h, k],
                    send_sem=send_sems.at[6 * h + k],
                    recv_sem=recv_sems.at[6 * h + k],
                    device_id=(p1,),
                    device_id_type=pl.DeviceIdType.MESH,
                )
                r.start()
                rds.append(r)
            return rds

        def finish_rs1_start_rs2(h, rds):
            p1, p2, _ = half_plan(h)
            for r in rds:
                r.wait()
            acc_ref[rows(my), cols(h)] += r1_ref[h, 0]
            acc_ref[rows(p2), cols(h)] += r1_ref[h, 1]
            r = pltpu.make_async_remote_copy(
                src_ref=acc_ref.at[rows(p2), cols(h)],
                dst_ref=r2_ref.at[h],
                send_sem=send_sems.at[6 * h + 2],
                recv_sem=recv_sems.at[6 * h + 2],
                device_id=(p2,),
                device_id_type=pl.DeviceIdType.MESH,
            )
            r.start()
            return [r]

        def finish_rs2_start_ag1(h, rds):
            p1, p2, _ = half_plan(h)
            for r in rds:
                r.wait()
            out_ref[rows(my), cols(h)] = (
                acc_ref[rows(my), cols(h)] + r2_ref[h]
            )
            r = pltpu.make_async_remote_copy(
                src_ref=out_ref.at[rows(my), cols(h)],
                dst_ref=out_ref.at[rows(my), cols(h)],
                send_sem=send_sems.at[6 * h + 3],
                recv_sem=recv_sems.at[6 * h + 3],
                device_id=(p2,),
                device_id_type=pl.DeviceIdType.MESH,
            )
            r.start()
            return [r]

        def finish_ag1_start_ag2(h, rds):
            p1, p2, _ = half_plan(h)
            for r in rds:
                r.wait()
            rds2 = []
            for k, q in enumerate((my, p2)):
                r = pltpu.make_async_remote_copy(
                    src_ref=out_ref.at[rows(q), cols(h)],
                    dst_ref=out_ref.at[rows(q), cols(h)],
                    send_sem=send_sems.at[6 * h + 4 + k],
                    recv_sem=recv_sems.at[6 * h + 4 + k],
                    device_id=(p1,),
                    device_id_type=pl.DeviceIdType.MESH,
                )
                r.start()
                rds2.append(r)
            return rds2

        p1_0, p2_0, send_qs_0 = half_plan(0)
        p1_1, p2_1, send_qs_1 = half_plan(1)
        compute_quarters(0, send_qs_0)
        h0 = start_rs1(0)
        compute_quarters(1, send_qs_1)
        h1 = start_rs1(1)
        compute_quarters(0, (my, p2_0))
        compute_quarters(1, (my, p2_1))
        h0 = finish_rs1_start_rs2(0, h0)
        h1 = finish_rs1_start_rs2(1, h1)
        h0 = finish_rs2_start_ag1(0, h0)
        h1 = finish_rs2_start_ag1(1, h1)
        h0 = finish_ag1_start_ag2(0, h0)
        h1 = finish_ag1_start_ag2(1, h1)
        for r in h0 + h1:
            r.wait()

    return pl.pallas_call(
        body,
        out_shape=jax.ShapeDtypeStruct((N_TOK, D_OUT), jnp.float32),
        in_specs=[
            pl.BlockSpec(memory_space=pltpu.VMEM),
            pl.BlockSpec(memory_space=pltpu.VMEM),
            pl.BlockSpec(memory_space=pltpu.VMEM),
            pl.BlockSpec(memory_space=pltpu.VMEM),
        ],
        out_specs=pl.BlockSpec(memory_space=pltpu.VMEM),
        scratch_shapes=[
            pltpu.VMEM((N_TOK, D_OUT), jnp.float32),
            pltpu.VMEM((2, 2, Q, HALF), jnp.float32),
            pltpu.VMEM((2, Q, HALF), jnp.float32),
            pltpu.VMEM((N_TOK, EXP_PER_DEV), jnp.float32),
            pltpu.SemaphoreType.DMA((12,)),
            pltpu.SemaphoreType.DMA((12,)),
        ],
        compiler_params=pltpu.CompilerParams(collective_id=0),
    )(x, router_W, route_idx, expert_W)
